# Initial kernel scaffold; baseline (speedup 1.0000x reference)
import jax
import jax.numpy as jnp
from jax.experimental import pallas as pl


def kernel(table, indices):
    raise NotImplementedError("write your pallas kernel here")


# speedup vs baseline: 2.3002x; 2.3002x over previous
import functools

import jax
import jax.numpy as jnp
from jax import lax
from jax.experimental import pallas as pl
from jax.experimental.pallas import tpu as pltpu
from jax.experimental.pallas import tpu_sc as plsc


def _gather_call(V, D, B):
    info = plsc.get_sparse_core_info()
    NC, NS = info.num_cores, info.num_subcores
    NW = NC * NS
    b_per_w = B // NW
    C = 256
    n_chunks = b_per_w // C
    REM = D - 256

    mesh = plsc.VectorSubcoreMesh(core_axis_name="c", subcore_axis_name="s")

    @functools.partial(
        pl.kernel,
        mesh=mesh,
        out_type=jax.ShapeDtypeStruct((B, D), jnp.float32),
        scratch_types=[
            pltpu.VMEM((C,), jnp.int32),
            pltpu.VMEM((C, 128), jnp.float32),
            pltpu.VMEM((C, 128), jnp.float32),
            pltpu.VMEM((C, REM), jnp.float32),
            pltpu.SemaphoreType.DMA,
            pltpu.SemaphoreType.DMA,
        ],
        compiler_params=pltpu.CompilerParams(needs_layout_passes=False),
    )
    def gather_kernel(table_hbm, idx_hbm, out_hbm, idx_v, b0_v, b1_v,
                      rem_v, sem, rsem):
        wid = lax.axis_index("s") * NC + lax.axis_index("c")
        base = wid * b_per_w
        lane = lax.iota(jnp.int32, 16)

        def chunk(g, carry):
            off = base + g * C
            pltpu.sync_copy(idx_hbm.at[pl.ds(off, C)], idx_v)
            cp0 = pltpu.async_copy(
                table_hbm.at[idx_v, pl.ds(0, 128)], b0_v, sem)
            cp1 = pltpu.async_copy(
                table_hbm.at[idx_v, pl.ds(128, 128)], b1_v, sem)

            def row16(i16, carry2):
                iv = idx_v[pl.ds(i16 * 16, 16)]
                for j in range(16):
                    r = jnp.max(jnp.where(lane == j, iv, 0))
                    pltpu.async_copy(
                        table_hbm.at[pl.ds(r, 1), pl.ds(256, REM)],
                        rem_v.at[pl.ds(i16 * 16 + j, 1), :], rsem)
                return carry2

            lax.fori_loop(0, C // 16, row16, 0)
            cp0.wait()
            cp1.wait()
            pltpu.sync_copy(b0_v, out_hbm.at[pl.ds(off, C), pl.ds(0, 128)])
            pltpu.sync_copy(b1_v, out_hbm.at[pl.ds(off, C), pl.ds(128, 128)])

            def drain(i, carry2):
                pltpu.make_async_copy(
                    table_hbm.at[pl.ds(0, 1), pl.ds(256, REM)],
                    rem_v.at[pl.ds(i, 1), :], rsem).wait()
                return carry2

            lax.fori_loop(0, C, drain, 0)
            pltpu.sync_copy(rem_v, out_hbm.at[pl.ds(off, C), pl.ds(256, REM)])
            return carry

        lax.fori_loop(0, n_chunks, chunk, 0)

    return gather_kernel


def kernel(table, indices):
    V, D = table.shape
    Bt, S = indices.shape
    B = Bt * S
    idx_flat = indices.reshape(B).astype(jnp.int32)
    out = _gather_call(V, D, B)(table, idx_flat)
    return out.reshape(Bt, S, D)


# speedup vs baseline: 2.5727x; 1.1185x over previous
import functools

import jax
import jax.numpy as jnp
from jax import lax
from jax.experimental import pallas as pl
from jax.experimental.pallas import tpu as pltpu
from jax.experimental.pallas import tpu_sc as plsc


def _gather_call(V, D, Bt, S):
    info = plsc.get_sparse_core_info()
    NC, NS = info.num_cores, info.num_subcores
    NW = NC * NS
    bt_per_w = Bt // NW
    NB = 4
    n_chunks = bt_per_w // NB
    SP = 64
    RP = 56
    REM = D - 256
    NG = S // 16
    TAIL = S - NG * 16

    mesh = plsc.VectorSubcoreMesh(core_axis_name="c", subcore_axis_name="s")

    @functools.partial(
        pl.kernel,
        mesh=mesh,
        out_type=jax.ShapeDtypeStruct((Bt, S, D), jnp.float32),
        scratch_types=[
            pltpu.VMEM((NB, SP), jnp.int32),
            pltpu.VMEM((NB, RP, 128), jnp.float32),
            pltpu.VMEM((NB, RP, 128), jnp.float32),
            pltpu.VMEM((NB, RP, REM), jnp.float32),
            pltpu.SemaphoreType.DMA,
            pltpu.SemaphoreType.DMA,
        ],
        compiler_params=pltpu.CompilerParams(needs_layout_passes=False),
    )
    def gather_kernel(table_hbm, idx_hbm, out_hbm, idx_v, b0_v, b1_v,
                      rem_v, sem, rsem):
        wid = lax.axis_index("s") * NC + lax.axis_index("c")
        base_b = wid * bt_per_w
        lane = lax.iota(jnp.int32, 16)

        def chunk(g, carry):
            bb = base_b + g * NB
            for k in range(NB):
                pltpu.sync_copy(idx_hbm.at[bb + k, :],
                                idx_v.at[k, pl.ds(0, S)])
            cps = []
            for k in range(NB):
                cps.append(pltpu.async_copy(
                    table_hbm.at[idx_v.at[k, pl.ds(0, S)], pl.ds(0, 128)],
                    b0_v.at[k, pl.ds(0, S), :], sem))
                cps.append(pltpu.async_copy(
                    table_hbm.at[idx_v.at[k, pl.ds(0, S)], pl.ds(128, 128)],
                    b1_v.at[k, pl.ds(0, S), :], sem))

            def row16(t, carry2):
                k = t // NG
                o = (t % NG) * 16
                iv = idx_v[k, pl.ds(o, 16)]
                for j in range(16):
                    r = jnp.max(jnp.where(lane == j, iv, 0))
                    pltpu.async_copy(
                        table_hbm.at[pl.ds(r, 1), pl.ds(256, REM)],
                        rem_v.at[k, pl.ds(o + j, 1), :], rsem)
                return carry2

            lax.fori_loop(0, NB * NG, row16, 0)
            for k in range(NB):
                iv = idx_v[k, pl.ds(NG * 16, 16)]
                for j in range(TAIL):
                    r = jnp.max(jnp.where(lane == j, iv, 0))
                    pltpu.async_copy(
                        table_hbm.at[pl.ds(r, 1), pl.ds(256, REM)],
                        rem_v.at[k, pl.ds(NG * 16 + j, 1), :], rsem)

            for cp in cps:
                cp.wait()
            for k in range(NB):
                pltpu.sync_copy(b0_v.at[k, pl.ds(0, S), :],
                                out_hbm.at[bb + k, :, pl.ds(0, 128)])
                pltpu.sync_copy(b1_v.at[k, pl.ds(0, S), :],
                                out_hbm.at[bb + k, :, pl.ds(128, 128)])

            def drain(i, carry2):
                pltpu.make_async_copy(
                    table_hbm.at[pl.ds(0, 1), pl.ds(256, REM)],
                    rem_v.at[0, pl.ds(0, 1), :], rsem).wait()
                return carry2

            lax.fori_loop(0, NB * S, drain, 0)
            for k in range(NB):
                pltpu.sync_copy(rem_v.at[k, pl.ds(0, S), :],
                                out_hbm.at[bb + k, :, pl.ds(256, REM)])
            return carry

        lax.fori_loop(0, n_chunks, chunk, 0)

    return gather_kernel


def kernel(table, indices):
    V, D = table.shape
    Bt, S = indices.shape
    idx = indices.astype(jnp.int32)
    return _gather_call(V, D, Bt, S)(table, idx)
